# baseline (speedup 1.0000x reference)
import functools

import jax
import jax.numpy as jnp
from jax import lax
from jax.experimental import pallas as pl
from jax.experimental.pallas import tpu as pltpu
from jax.experimental.pallas import tpu_sc as plsc

DIM = 512
NN = 8
NRELP1 = 17
NC = 2
NS = 16
NW = NC * NS
L = 16


def _sc_gather_user(u, usr):
    B = u.shape[0]
    bpw = B // NW
    mesh = plsc.VectorSubcoreMesh(core_axis_name="c", subcore_axis_name="s")

    @functools.partial(
        pl.kernel,
        out_type=jax.ShapeDtypeStruct((B, DIM), jnp.float32),
        mesh=mesh,
        scratch_types=[
            pltpu.VMEM((bpw,), jnp.int32),
            pltpu.VMEM((bpw, DIM), jnp.float32),
            pltpu.SemaphoreType.DMA,
        ],
    )
    def k(u_hbm, usr_hbm, ue_out, idxu, rows, sem):
        wid = lax.axis_index("s") * NC + lax.axis_index("c")
        base = wid * bpw
        pltpu.sync_copy(u_hbm.at[pl.ds(base, bpw)], idxu)
        pltpu.async_copy(usr_hbm.at[idxu], rows, sem).wait()
        pltpu.sync_copy(rows, ue_out.at[pl.ds(base, bpw)])

    return k(u, usr)


def _sc_gather_adj(v, ae128, ar128):
    B = v.shape[0]
    bpw = B // NW
    GP = 128 // NN
    mesh = plsc.VectorSubcoreMesh(core_axis_name="c", subcore_axis_name="s")

    @functools.partial(
        pl.kernel,
        out_type=[
            jax.ShapeDtypeStruct((B * NN,), jnp.int32),
            jax.ShapeDtypeStruct((B * NN,), jnp.int32),
        ],
        mesh=mesh,
        scratch_types=[
            pltpu.VMEM((bpw,), jnp.int32),
            pltpu.VMEM((bpw,), jnp.int32),
            pltpu.VMEM((bpw, 128), jnp.int32),
            pltpu.VMEM((bpw, 128), jnp.int32),
            pltpu.VMEM((bpw * NN,), jnp.int32),
            pltpu.VMEM((bpw * NN,), jnp.int32),
            pltpu.SemaphoreType.DMA,
        ],
        compiler_params=pltpu.CompilerParams(needs_layout_passes=False),
    )
    def k(v_hbm, ae_hbm, ar_hbm, e1_out, r0_out,
          idxv, idxg, gae, gar, e1f, r0f, sem):
        wid = lax.axis_index("s") * NC + lax.axis_index("c")
        base = wid * bpw
        lanes = jnp.arange(L, dtype=jnp.int32)
        pltpu.sync_copy(v_hbm.at[pl.ds(base, bpw)], idxv)
        for cc in range(bpw // L):
            sl = pl.ds(cc * L, L)
            idxg[sl] = lax.shift_right_logical(idxv[sl], 4)
        c2 = pltpu.async_copy(ae_hbm.at[idxg], gae, sem)
        c3 = pltpu.async_copy(ar_hbm.at[idxg], gar, sem)
        c2.wait()
        c3.wait()
        for g in range(bpw // 2):
            rowvec = jnp.full((L,), 2 * g, jnp.int32) + \
                lax.shift_right_logical(lanes, 3)
            vv = plsc.load_gather(idxv, [rowvec])
            colvec = (vv & (GP - 1)) * NN + (lanes & (NN - 1))
            e1f[pl.ds(g * L, L)] = plsc.load_gather(gae, [rowvec, colvec])
            r0f[pl.ds(g * L, L)] = plsc.load_gather(gar, [rowvec, colvec])
        pltpu.sync_copy(e1f, e1_out.at[pl.ds(base * NN, bpw * NN)])
        pltpu.sync_copy(r0f, r0_out.at[pl.ds(base * NN, bpw * NN)])

    return k(v, ae128, ar128)


def _tc_weights(user_emb, rel, rel0):
    B = user_emb.shape[0]
    blk = 1024

    def body(ue_ref, rel_ref, r0_ref, p_ref):
        ue = ue_ref[...]
        rs = lax.dot_general(ue, rel_ref[...], (((1,), (1,)), ((), ())),
                             preferred_element_type=jnp.float32)
        r0 = r0_ref[...]
        praw = jnp.zeros((blk, NN), jnp.float32)
        for kk in range(NRELP1):
            praw = jnp.where(r0 == kk, rs[:, kk][:, None], praw)
        m = jnp.max(praw, axis=1, keepdims=True)
        e = jnp.exp(praw - m)
        p_ref[...] = e / jnp.sum(e, axis=1, keepdims=True)

    return pl.pallas_call(
        body,
        grid=(B // blk,),
        in_specs=[
            pl.BlockSpec((blk, DIM), lambda i: (i, 0)),
            pl.BlockSpec((NRELP1, DIM), lambda i: (0, 0)),
            pl.BlockSpec((blk, NN), lambda i: (i, 0)),
        ],
        out_specs=pl.BlockSpec((blk, NN), lambda i: (i, 0)),
        out_shape=jax.ShapeDtypeStruct((B, NN), jnp.float32),
    )(user_emb, rel, rel0)


def _sc_agg(v, ent, ent1, p):
    B = v.shape[0]
    bpw = B // NW
    C = 16
    NCH = bpw // C
    mesh = plsc.VectorSubcoreMesh(core_axis_name="c", subcore_axis_name="s")

    @functools.partial(
        pl.kernel,
        out_type=jax.ShapeDtypeStruct((B, DIM), jnp.float32),
        mesh=mesh,
        scratch_types=[
            pltpu.VMEM((C, NN), jnp.int32),
            pltpu.VMEM((C,), jnp.int32),
            pltpu.VMEM((C, NN), jnp.float32),
            pltpu.VMEM((NN * C, DIM), jnp.float32),
            pltpu.VMEM((C, DIM), jnp.float32),
            pltpu.VMEM((C, DIM), jnp.float32),
            pltpu.SemaphoreType.DMA,
        ],
        compiler_params=pltpu.CompilerParams(needs_layout_passes=False),
    )
    def k(v_hbm, ent_hbm, e1_hbm, p_hbm, out_hbm,
          e1v, idxs, wv, nrows, srows, acc, sem):
        wid = lax.axis_index("s") * NC + lax.axis_index("c")
        base = wid * bpw
        lanes = jnp.arange(L, dtype=jnp.int32)

        def chunk(ch, carry):
            rowbase = base + ch * C
            pltpu.sync_copy(e1_hbm.at[pl.ds(rowbase, C)], e1v)
            pltpu.sync_copy(p_hbm.at[pl.ds(rowbase, C)], wv)
            pltpu.sync_copy(v_hbm.at[pl.ds(rowbase, C)], idxs)
            cps = []
            for kk in range(NN):
                idx_vec = plsc.load_gather(
                    e1v, [lanes, jnp.full((L,), kk, jnp.int32)])
                cps.append(pltpu.async_copy(
                    ent_hbm.at[idx_vec], nrows.at[pl.ds(kk * C, C)], sem))
            cps.append(pltpu.async_copy(ent_hbm.at[idxs], srows, sem))
            for cp in cps:
                cp.wait()

            def row(r, carry2):
                rr = jnp.full((L,), r, jnp.int32)
                wbc = [plsc.load_gather(wv, [rr, jnp.full((L,), kk, jnp.int32)])
                       for kk in range(NN)]
                for cc in range(DIM // L):
                    sl = pl.ds(cc * L, L)
                    a = srows[r, sl]
                    for kk in range(NN):
                        a = a + wbc[kk] * nrows[kk * C + r, sl]
                    acc[r, sl] = a
                return carry2

            lax.fori_loop(0, C, row, 0)
            pltpu.sync_copy(acc, out_hbm.at[pl.ds(rowbase, C)])
            return carry

        lax.fori_loop(0, NCH, chunk, 0)

    return k(v, ent, ent1, p)


def _tc_final(user_emb, sv_agg, W, b2d):
    B = user_emb.shape[0]
    blk = 512

    def body(ue_ref, sv_ref, w_ref, b_ref, c_ref, s_ref):
        h = lax.dot_general(sv_ref[...], w_ref[...], (((1,), (1,)), ((), ())),
                            preferred_element_type=jnp.float32)
        item = jnp.tanh(h + b_ref[...])
        c_ref[...] = item[:, None, :]
        s = jnp.sum(ue_ref[...] * item, axis=1, keepdims=True)
        s_ref[...] = jax.nn.sigmoid(s)

    return pl.pallas_call(
        body,
        grid=(B // blk,),
        in_specs=[
            pl.BlockSpec((blk, DIM), lambda i: (i, 0)),
            pl.BlockSpec((blk, DIM), lambda i: (i, 0)),
            pl.BlockSpec((DIM, DIM), lambda i: (0, 0)),
            pl.BlockSpec((1, DIM), lambda i: (0, 0)),
        ],
        out_specs=[
            pl.BlockSpec((blk, 1, DIM), lambda i: (i, 0, 0)),
            pl.BlockSpec((blk, 1), lambda i: (i, 0)),
        ],
        out_shape=[
            jax.ShapeDtypeStruct((B, 1, DIM), jnp.float32),
            jax.ShapeDtypeStruct((B, 1), jnp.float32),
        ],
    )(user_emb, sv_agg, W, b2d)


def kernel(u, v, usr, ent, rel, adj_ent, adj_rel, W, b):
    B = u.shape[0]
    user_emb = _sc_gather_user(u, usr)
    e1f, r0f = _sc_gather_adj(v, adj_ent.reshape(-1, 128),
                              adj_rel.reshape(-1, 128))
    ent1 = e1f.reshape(B, NN)
    rel0 = r0f.reshape(B, NN)
    p = _tc_weights(user_emb, rel, rel0)
    sv_agg = _sc_agg(v, ent, ent1, p)
    c3d, s2d = _tc_final(user_emb, sv_agg, W, b.reshape(1, DIM))
    return (s2d.reshape(B), c3d, v.reshape(B, 1), ent1, rel0)


# speedup vs baseline: 1.3415x; 1.3415x over previous
import functools

import jax
import jax.numpy as jnp
from jax import lax
from jax.experimental import pallas as pl
from jax.experimental.pallas import tpu as pltpu
from jax.experimental.pallas import tpu_sc as plsc

DIM = 512
NN = 8
NRELP1 = 17
NC = 2
NS = 16
NW = NC * NS
L = 16


def _sc_gather(u, v, usr, adj_ent, adj_rel):
    B = u.shape[0]
    bpw = B // NW
    mesh = plsc.VectorSubcoreMesh(core_axis_name="c", subcore_axis_name="s")

    @functools.partial(
        pl.kernel,
        out_type=[
            jax.ShapeDtypeStruct((B, DIM), jnp.float32),
            jax.ShapeDtypeStruct((B, NN), jnp.int32),
            jax.ShapeDtypeStruct((B, NN), jnp.int32),
        ],
        mesh=mesh,
        scratch_types=[
            pltpu.VMEM((bpw,), jnp.int32),
            pltpu.VMEM((bpw,), jnp.int32),
            pltpu.VMEM((bpw, DIM), jnp.float32),
            pltpu.VMEM((bpw, NN), jnp.int32),
            pltpu.VMEM((bpw, NN), jnp.int32),
            pltpu.SemaphoreType.DMA,
            pltpu.SemaphoreType.DMA,
        ],
    )
    def k(u_hbm, v_hbm, usr_hbm, ae_hbm, ar_hbm, ue_out, e1_out, r0_out,
          idxu, idxv, rows, gae, gar, sem, sem2):
        wid = lax.axis_index("s") * NC + lax.axis_index("c")
        base = wid * bpw
        pltpu.sync_copy(u_hbm.at[pl.ds(base, bpw)], idxu)
        pltpu.sync_copy(v_hbm.at[pl.ds(base, bpw)], idxv)
        c1 = pltpu.async_copy(usr_hbm.at[idxu], rows, sem)
        K = 16
        for b in range(bpw // K):
            vv = idxv[pl.ds(b * K, K)]
            cps = []
            for j in range(K):
                i = b * K + j
                vi = vv[j]
                cps.append(pltpu.async_copy(
                    ae_hbm.at[pl.ds(vi, 1)], gae.at[pl.ds(i, 1)], sem2))
                cps.append(pltpu.async_copy(
                    ar_hbm.at[pl.ds(vi, 1)], gar.at[pl.ds(i, 1)], sem2))
            for cp in cps:
                cp.wait()
        c1.wait()
        pltpu.sync_copy(rows, ue_out.at[pl.ds(base, bpw)])
        pltpu.sync_copy(gae, e1_out.at[pl.ds(base, bpw)])
        pltpu.sync_copy(gar, r0_out.at[pl.ds(base, bpw)])

    return k(u, v, usr, adj_ent, adj_rel)


def _tc_weights(user_emb, rel, rel0):
    B = user_emb.shape[0]
    blk = 1024

    def body(ue_ref, rel_ref, r0_ref, p_ref):
        ue = ue_ref[...]
        rs = lax.dot_general(ue, rel_ref[...], (((1,), (1,)), ((), ())),
                             preferred_element_type=jnp.float32)
        r0 = r0_ref[...]
        praw = jnp.zeros((blk, NN), jnp.float32)
        for kk in range(NRELP1):
            praw = jnp.where(r0 == kk, rs[:, kk][:, None], praw)
        m = jnp.max(praw, axis=1, keepdims=True)
        e = jnp.exp(praw - m)
        p_ref[...] = e / jnp.sum(e, axis=1, keepdims=True)

    return pl.pallas_call(
        body,
        grid=(B // blk,),
        in_specs=[
            pl.BlockSpec((blk, DIM), lambda i: (i, 0)),
            pl.BlockSpec((NRELP1, DIM), lambda i: (0, 0)),
            pl.BlockSpec((blk, NN), lambda i: (i, 0)),
        ],
        out_specs=pl.BlockSpec((blk, NN), lambda i: (i, 0)),
        out_shape=jax.ShapeDtypeStruct((B, NN), jnp.float32),
    )(user_emb, rel, rel0)


def _sc_agg(v, ent, ent1, p):
    B = v.shape[0]
    bpw = B // NW
    C = 16
    NCH = bpw // C
    mesh = plsc.VectorSubcoreMesh(core_axis_name="c", subcore_axis_name="s")

    @functools.partial(
        pl.kernel,
        out_type=jax.ShapeDtypeStruct((B, DIM), jnp.float32),
        mesh=mesh,
        scratch_types=[
            pltpu.VMEM((C, NN), jnp.int32),
            pltpu.VMEM((C,), jnp.int32),
            pltpu.VMEM((C, NN), jnp.float32),
            pltpu.VMEM((NN * C, DIM), jnp.float32),
            pltpu.VMEM((C, DIM), jnp.float32),
            pltpu.VMEM((C, DIM), jnp.float32),
            pltpu.SemaphoreType.DMA,
        ],
        compiler_params=pltpu.CompilerParams(needs_layout_passes=False),
    )
    def k(v_hbm, ent_hbm, e1_hbm, p_hbm, out_hbm,
          e1v, idxs, wv, nrows, srows, acc, sem):
        wid = lax.axis_index("s") * NC + lax.axis_index("c")
        base = wid * bpw
        lanes = jnp.arange(L, dtype=jnp.int32)

        def chunk(ch, carry):
            rowbase = base + ch * C
            pltpu.sync_copy(e1_hbm.at[pl.ds(rowbase, C)], e1v)
            pltpu.sync_copy(p_hbm.at[pl.ds(rowbase, C)], wv)
            pltpu.sync_copy(v_hbm.at[pl.ds(rowbase, C)], idxs)
            cps = []
            for kk in range(NN):
                idx_vec = plsc.load_gather(
                    e1v, [lanes, jnp.full((L,), kk, jnp.int32)])
                cps.append(pltpu.async_copy(
                    ent_hbm.at[idx_vec], nrows.at[pl.ds(kk * C, C)], sem))
            cps.append(pltpu.async_copy(ent_hbm.at[idxs], srows, sem))
            for cp in cps:
                cp.wait()

            def row(r, carry2):
                rr = jnp.full((L,), r, jnp.int32)
                wbc = [plsc.load_gather(wv, [rr, jnp.full((L,), kk, jnp.int32)])
                       for kk in range(NN)]
                for cc in range(DIM // L):
                    sl = pl.ds(cc * L, L)
                    a = srows[r, sl]
                    for kk in range(NN):
                        a = a + wbc[kk] * nrows[kk * C + r, sl]
                    acc[r, sl] = a
                return carry2

            lax.fori_loop(0, C, row, 0)
            pltpu.sync_copy(acc, out_hbm.at[pl.ds(rowbase, C)])
            return carry

        lax.fori_loop(0, NCH, chunk, 0)

    return k(v, ent, ent1, p)


def _tc_final(user_emb, sv_agg, W, b2d):
    B = user_emb.shape[0]
    blk = 512

    def body(ue_ref, sv_ref, w_ref, b_ref, c_ref, s_ref):
        h = lax.dot_general(sv_ref[...], w_ref[...], (((1,), (1,)), ((), ())),
                            preferred_element_type=jnp.float32)
        item = jnp.tanh(h + b_ref[...])
        c_ref[...] = item[:, None, :]
        s = jnp.sum(ue_ref[...] * item, axis=1, keepdims=True)
        s_ref[...] = jax.nn.sigmoid(s)

    return pl.pallas_call(
        body,
        grid=(B // blk,),
        in_specs=[
            pl.BlockSpec((blk, DIM), lambda i: (i, 0)),
            pl.BlockSpec((blk, DIM), lambda i: (i, 0)),
            pl.BlockSpec((DIM, DIM), lambda i: (0, 0)),
            pl.BlockSpec((1, DIM), lambda i: (0, 0)),
        ],
        out_specs=[
            pl.BlockSpec((blk, 1, DIM), lambda i: (i, 0, 0)),
            pl.BlockSpec((blk, 1), lambda i: (i, 0)),
        ],
        out_shape=[
            jax.ShapeDtypeStruct((B, 1, DIM), jnp.float32),
            jax.ShapeDtypeStruct((B, 1), jnp.float32),
        ],
    )(user_emb, sv_agg, W, b2d)


def kernel(u, v, usr, ent, rel, adj_ent, adj_rel, W, b):
    B = u.shape[0]
    user_emb, ent1, rel0 = _sc_gather(u, v, usr, adj_ent, adj_rel)
    p = _tc_weights(user_emb, rel, rel0)
    sv_agg = _sc_agg(v, ent, ent1, p)
    c3d, s2d = _tc_final(user_emb, sv_agg, W, b.reshape(1, DIM))
    return (s2d.reshape(B), c3d, v.reshape(B, 1), ent1, rel0)
